# Initial kernel scaffold; baseline (speedup 1.0000x reference)
import jax
import jax.numpy as jnp
from jax.experimental import pallas as pl


def kernel(x, edge_index, batch, W1, b1, g1, be1, W2, b2, g2, be2, W3, b3):
    raise NotImplementedError("write your pallas kernel here")


# speedup vs baseline: 2.1517x; 2.1517x over previous
import dataclasses
import functools

import jax
import jax.numpy as jnp
from jax import lax
from jax.experimental import pallas as pl
from jax.experimental.pallas import tpu as pltpu
from jax.experimental.pallas import tpu_sc as plsc

MAX_NODES = 28
EPS = 1e-5
G = 512
N_NODES = 4096
E_EDGES = 32768
D_FEAT = 256
A_ROW = 1024
A_WORDS = G * A_ROW
X_ROWS = G * MAX_NODES
N_PAD_ROWS = 256

NC, NS = 2, 16
NW = NC * NS
XR_PER_TILE = X_ROWS // NW
X_CHUNK = 112
E_PER_TILE = E_EDGES // NS


def _densify_body(x_hbm, ei_hbm, batch_hbm, a_hbm, xs_hbm,
                  batch_v, counts_v, starts_v, rows_buf, xidx_v,
                  src_v, dst_v, eidx2d, ones_v, zbuf, a_sh, sem):
    cid = lax.axis_index("c")
    sid = lax.axis_index("s")
    wid = cid * NS + sid

    pltpu.sync_copy(batch_hbm, batch_v)

    @pl.loop(0, G // 16)
    def _zero_counts(j):
        counts_v[pl.ds(j * 16, 16)] = jnp.zeros((16,), jnp.int32)

    ones16 = jnp.ones((16,), jnp.int32)

    @pl.loop(0, N_NODES // 16)
    def _hist(j):
        b = batch_v[pl.ds(j * 16, 16)]
        plsc.addupdate_scatter(counts_v, [b], ones16)

    def _scan_body(j, carry):
        c = counts_v[pl.ds(j * 16, 16)]
        incl = plsc.cumsum(c)
        starts_v[pl.ds(j * 16, 16)] = incl - c + carry
        return carry + jnp.sum(c)

    lax.fori_loop(0, G // 16, _scan_body, jnp.int32(0))

    @pl.loop(0, 8192 // 16)
    def _zero_zbuf(j):
        zbuf[pl.ds(j * 16, 16)] = jnp.zeros((16,), jnp.float32)

    @pl.loop(0, 128 // 16)
    def _init_ones(j):
        ones_v[pl.ds(j * 16, 16)] = jnp.ones((16,), jnp.float32)

    @pl.when(cid == 0)
    def _zero_spmem():
        for q in range(4):
            pltpu.sync_copy(zbuf, a_sh.at[pl.ds(sid * 32768 + q * 8192, 8192)])

    plsc.subcore_barrier()

    @pl.when(cid == 0)
    def _edges():
        pltpu.sync_copy(ei_hbm.at[0, pl.ds(sid * E_PER_TILE, E_PER_TILE)],
                        src_v)
        pltpu.sync_copy(ei_hbm.at[1, pl.ds(sid * E_PER_TILE, E_PER_TILE)],
                        dst_v)

        @pl.loop(0, 16)
        def _row(r):
            @pl.loop(0, 8)
            def _chunk(cc):
                j = r * 8 + cc
                s = src_v[pl.ds(j * 16, 16)]
                d = dst_v[pl.ds(j * 16, 16)]
                g = plsc.load_gather(batch_v, [s])
                st = plsc.load_gather(starts_v, [g])
                ls = s - st
                ld = d - st
                valid = (ls < MAX_NODES) & (ld >= 0) & (ld < MAX_NODES)
                flat = g * A_ROW + ls * MAX_NODES + ld
                dump = (s & 511) * A_ROW + 784 + (d & 127)
                eidx2d[r, pl.ds(cc * 16, 16)] = jnp.where(valid, flat, dump)

            pltpu.sync_copy(ones_v, a_sh.at[eidx2d.at[r]], add=True)

    plsc.subcore_barrier()

    @pl.when(cid == 0)
    def _copy_a():
        pltpu.sync_copy(a_sh.at[pl.ds(sid * 32768, 32768)],
                        a_hbm.at[pl.ds(sid * 32768, 32768)])

    base = wid * XR_PER_TILE
    for c in range(XR_PER_TILE // X_CHUNK):
        cbase = base + c * X_CHUNK

        @pl.loop(0, X_CHUNK // 16)
        def _mkidx(i):
            r = cbase + i * 16 + lax.iota(jnp.int32, 16)
            gg = r // MAX_NODES
            ll = r - gg * MAX_NODES
            st = plsc.load_gather(starts_v, [gg])
            cnt = plsc.load_gather(counts_v, [gg])
            srcp = st + ll
            sentinel = N_NODES + (r & (N_PAD_ROWS - 1))
            xidx_v[pl.ds(i * 16, 16)] = jnp.where(ll < cnt, srcp, sentinel)

        pltpu.async_copy(x_hbm.at[xidx_v], rows_buf, sem).wait()
        pltpu.sync_copy(rows_buf, xs_hbm.at[pl.ds(cbase, X_CHUNK)])


def _sc_compiler_params():
    cp = pltpu.CompilerParams()
    if "needs_layout_passes" in pltpu.CompilerParams.__dataclass_fields__:
        cp = dataclasses.replace(cp, needs_layout_passes=False)
    return cp


def _densify(x_aug, edge_index, batch):
    mesh = plsc.VectorSubcoreMesh(core_axis_name="c", subcore_axis_name="s")
    kern = pl.kernel(
        _densify_body,
        out_type=(
            jax.ShapeDtypeStruct((A_WORDS,), jnp.float32),
            jax.ShapeDtypeStruct((X_ROWS, D_FEAT), jnp.float32),
        ),
        mesh=mesh,
        scratch_types=[
            pltpu.VMEM((N_NODES,), jnp.int32),
            pltpu.VMEM((G,), jnp.int32),
            pltpu.VMEM((G,), jnp.int32),
            pltpu.VMEM((X_CHUNK, D_FEAT), jnp.float32),
            pltpu.VMEM((X_CHUNK,), jnp.int32),
            pltpu.VMEM((E_PER_TILE,), jnp.int32),
            pltpu.VMEM((E_PER_TILE,), jnp.int32),
            pltpu.VMEM((16, 128), jnp.int32),
            pltpu.VMEM((128,), jnp.float32),
            pltpu.VMEM((8192,), jnp.float32),
            pltpu.VMEM_SHARED((A_WORDS,), jnp.float32),
            pltpu.SemaphoreType.DMA,
        ],
        compiler_params=_sc_compiler_params(),
    )
    return kern(x_aug, edge_index, batch)


def _bn_relu_epilogue(acc, b_ref, g_ref, be_ref):
    h = acc + b_ref[...]
    mu = jnp.mean(h, axis=0, keepdims=True)
    hc = h - mu
    var = jnp.mean(hc * hc, axis=0, keepdims=True)
    return jnp.maximum(hc * lax.rsqrt(var + EPS) * g_ref[...] + be_ref[...],
                       0.0)


def _l1_body(a_ref, x_ref, wa_ref, wx_ref, b_ref, g_ref, be_ref, o_ref):
    k = pl.program_id(1)

    @pl.when(k == 0)
    def _():
        o_ref[...] = jnp.dot(a_ref[...], wa_ref[...],
                             preferred_element_type=jnp.float32)

    o_ref[...] += jnp.dot(x_ref[...], wx_ref[...],
                          preferred_element_type=jnp.float32)

    @pl.when(k == pl.num_programs(1) - 1)
    def _():
        o_ref[...] = _bn_relu_epilogue(o_ref[...], b_ref, g_ref, be_ref)


def _layer1(a, xs, w1a, w1x, b1, g1, be1):
    M, NB, KB = 512, 512, 1024
    nn, nk = 4096 // NB, 7168 // KB
    return pl.pallas_call(
        _l1_body,
        grid=(nn, nk),
        in_specs=[
            pl.BlockSpec((M, 784), lambda n, k: (0, 0)),
            pl.BlockSpec((M, KB), lambda n, k: (0, k)),
            pl.BlockSpec((784, NB), lambda n, k: (0, n)),
            pl.BlockSpec((KB, NB), lambda n, k: (k, n)),
            pl.BlockSpec((1, NB), lambda n, k: (0, n)),
            pl.BlockSpec((1, NB), lambda n, k: (0, n)),
            pl.BlockSpec((1, NB), lambda n, k: (0, n)),
        ],
        out_specs=pl.BlockSpec((M, NB), lambda n, k: (0, n)),
        out_shape=jax.ShapeDtypeStruct((M, 4096), jnp.float32),
        compiler_params=pltpu.CompilerParams(
            dimension_semantics=("parallel", "arbitrary")),
    )(a, xs, w1a, w1x, b1, g1, be1)


def _l23_body(x_ref, w_ref, b_ref, g_ref, be_ref, o_ref, *, bn):
    k = pl.program_id(1)

    @pl.when(k == 0)
    def _():
        o_ref[...] = jnp.zeros_like(o_ref)

    o_ref[...] += jnp.dot(x_ref[...], w_ref[...],
                          preferred_element_type=jnp.float32)

    @pl.when(k == pl.num_programs(1) - 1)
    def _():
        if bn:
            o_ref[...] = _bn_relu_epilogue(o_ref[...], b_ref, g_ref, be_ref)
        else:
            o_ref[...] = o_ref[...] + b_ref[...]


def _layer23(h, w, b, g, be, bn):
    M, NB, KB = 512, 512, 1024
    K, NO = w.shape
    nn, nk = NO // NB, K // KB
    specs = [
        pl.BlockSpec((M, KB), lambda n, k: (0, k)),
        pl.BlockSpec((KB, NB), lambda n, k: (k, n)),
        pl.BlockSpec((1, NB), lambda n, k: (0, n)),
        pl.BlockSpec((1, NB), lambda n, k: (0, n)),
        pl.BlockSpec((1, NB), lambda n, k: (0, n)),
    ]
    return pl.pallas_call(
        functools.partial(_l23_body, bn=bn),
        grid=(nn, nk),
        in_specs=specs,
        out_specs=pl.BlockSpec((M, NB), lambda n, k: (0, n)),
        out_shape=jax.ShapeDtypeStruct((M, NO), jnp.float32),
        compiler_params=pltpu.CompilerParams(
            dimension_semantics=("parallel", "arbitrary")),
    )(h, w, b, g, be)


def kernel(x, edge_index, batch, W1, b1, g1, be1, W2, b2, g2, be2, W3, b3):
    x_aug = jnp.concatenate(
        [x, jnp.zeros((N_PAD_ROWS, D_FEAT), x.dtype)], axis=0)
    a_flat, xs = _densify(x_aug, edge_index, batch)
    a = a_flat.reshape(G, A_ROW)[:, :784]
    xrows = xs.reshape(G, MAX_NODES * D_FEAT)

    w1a = W1[:784]
    w1x = W1[784:]
    h1 = _layer1(a, xrows, w1a, w1x, b1.reshape(1, -1), g1.reshape(1, -1),
                 be1.reshape(1, -1))
    h2 = _layer23(h1, W2, b2.reshape(1, -1), g2.reshape(1, -1),
                  be2.reshape(1, -1), bn=True)
    return _layer23(h2, W3, b3.reshape(1, -1), b3.reshape(1, -1),
                    b3.reshape(1, -1), bn=False)


# speedup vs baseline: 2.2861x; 1.0624x over previous
import dataclasses
import functools

import jax
import jax.numpy as jnp
from jax import lax
from jax.experimental import pallas as pl
from jax.experimental.pallas import tpu as pltpu
from jax.experimental.pallas import tpu_sc as plsc

MAX_NODES = 28
EPS = 1e-5
G = 512
N_NODES = 4096
E_EDGES = 32768
D_FEAT = 256
A_ROW = 1024
A_WORDS = G * A_ROW
X_ROWS = G * MAX_NODES
N_PAD_ROWS = 256

NC, NS = 2, 16
NW = NC * NS
XR_PER_TILE = X_ROWS // NW
X_CHUNK = 112
E_PER_TILE = E_EDGES // NS


def _densify_body(x_hbm, ei_hbm, batch_hbm, a_hbm, xs_hbm,
                  batch_v, counts_v, starts_v, rows_buf, xidx_v,
                  src_v, dst_v, eidx2d, ones_v, zbuf, a_sh, sem):
    cid = lax.axis_index("c")
    sid = lax.axis_index("s")
    wid = cid * NS + sid

    pltpu.sync_copy(batch_hbm, batch_v)

    @pl.loop(0, G // 16)
    def _zero_counts(j):
        counts_v[pl.ds(j * 16, 16)] = jnp.zeros((16,), jnp.int32)

    ones16 = jnp.ones((16,), jnp.int32)

    @pl.loop(0, N_NODES // 16)
    def _hist(j):
        b = batch_v[pl.ds(j * 16, 16)]
        plsc.addupdate_scatter(counts_v, [b], ones16)

    def _scan_body(j, carry):
        c = counts_v[pl.ds(j * 16, 16)]
        incl = plsc.cumsum(c)
        starts_v[pl.ds(j * 16, 16)] = incl - c + carry
        return carry + jnp.sum(c)

    lax.fori_loop(0, G // 16, _scan_body, jnp.int32(0))

    @pl.loop(0, 8192 // 16)
    def _zero_zbuf(j):
        zbuf[pl.ds(j * 16, 16)] = jnp.zeros((16,), jnp.float32)

    @pl.loop(0, 128 // 16)
    def _init_ones(j):
        ones_v[pl.ds(j * 16, 16)] = jnp.ones((16,), jnp.float32)

    @pl.when(cid == 0)
    def _zero_spmem():
        for q in range(4):
            pltpu.sync_copy(zbuf, a_sh.at[pl.ds(sid * 32768 + q * 8192, 8192)])

    plsc.subcore_barrier()

    @pl.when(cid == 0)
    def _edges():
        pltpu.sync_copy(ei_hbm.at[0, pl.ds(sid * E_PER_TILE, E_PER_TILE)],
                        src_v)
        pltpu.sync_copy(ei_hbm.at[1, pl.ds(sid * E_PER_TILE, E_PER_TILE)],
                        dst_v)

        @pl.loop(0, 16)
        def _row(r):
            @pl.loop(0, 8)
            def _chunk(cc):
                j = r * 8 + cc
                s = src_v[pl.ds(j * 16, 16)]
                d = dst_v[pl.ds(j * 16, 16)]
                g = plsc.load_gather(batch_v, [s])
                st = plsc.load_gather(starts_v, [g])
                ls = s - st
                ld = d - st
                valid = (ls < MAX_NODES) & (ld >= 0) & (ld < MAX_NODES)
                flat = g * A_ROW + ls * MAX_NODES + ld
                dump = (s & 511) * A_ROW + 784 + (d & 127)
                eidx2d[r, pl.ds(cc * 16, 16)] = jnp.where(valid, flat, dump)

            pltpu.sync_copy(ones_v, a_sh.at[eidx2d.at[r]], add=True)

    plsc.subcore_barrier()

    @pl.when(cid == 0)
    def _copy_a():
        pltpu.sync_copy(a_sh.at[pl.ds(sid * 32768, 32768)],
                        a_hbm.at[pl.ds(sid * 32768, 32768)])

    base = wid * XR_PER_TILE
    for c in range(XR_PER_TILE // X_CHUNK):
        cbase = base + c * X_CHUNK

        @pl.loop(0, X_CHUNK // 16)
        def _mkidx(i):
            r = cbase + i * 16 + lax.iota(jnp.int32, 16)
            gg = r // MAX_NODES
            ll = r - gg * MAX_NODES
            st = plsc.load_gather(starts_v, [gg])
            cnt = plsc.load_gather(counts_v, [gg])
            srcp = st + ll
            sentinel = N_NODES + (r & (N_PAD_ROWS - 1))
            xidx_v[pl.ds(i * 16, 16)] = jnp.where(ll < cnt, srcp, sentinel)

        pltpu.async_copy(x_hbm.at[xidx_v], rows_buf, sem).wait()
        pltpu.sync_copy(rows_buf, xs_hbm.at[pl.ds(cbase, X_CHUNK)])


def _sc_compiler_params():
    cp = pltpu.CompilerParams()
    if "needs_layout_passes" in pltpu.CompilerParams.__dataclass_fields__:
        cp = dataclasses.replace(cp, needs_layout_passes=False)
    return cp


def _densify(x_aug, edge_index, batch):
    mesh = plsc.VectorSubcoreMesh(core_axis_name="c", subcore_axis_name="s")
    kern = pl.kernel(
        _densify_body,
        out_type=(
            jax.ShapeDtypeStruct((A_WORDS,), jnp.float32),
            jax.ShapeDtypeStruct((X_ROWS, D_FEAT), jnp.float32),
        ),
        mesh=mesh,
        scratch_types=[
            pltpu.VMEM((N_NODES,), jnp.int32),
            pltpu.VMEM((G,), jnp.int32),
            pltpu.VMEM((G,), jnp.int32),
            pltpu.VMEM((X_CHUNK, D_FEAT), jnp.float32),
            pltpu.VMEM((X_CHUNK,), jnp.int32),
            pltpu.VMEM((E_PER_TILE,), jnp.int32),
            pltpu.VMEM((E_PER_TILE,), jnp.int32),
            pltpu.VMEM((16, 128), jnp.int32),
            pltpu.VMEM((128,), jnp.float32),
            pltpu.VMEM((8192,), jnp.float32),
            pltpu.VMEM_SHARED((A_WORDS,), jnp.float32),
            pltpu.SemaphoreType.DMA,
        ],
        compiler_params=_sc_compiler_params(),
    )
    return kern(x_aug, edge_index, batch)


def _bn_relu_epilogue(acc, b_ref, g_ref, be_ref):
    h = acc + b_ref[...]
    mu = jnp.mean(h, axis=0, keepdims=True)
    hc = h - mu
    var = jnp.mean(hc * hc, axis=0, keepdims=True)
    return jnp.maximum(hc * lax.rsqrt(var + EPS) * g_ref[...] + be_ref[...],
                       0.0)


def _l1_body(a_ref, x_ref, w_ref, b_ref, g_ref, be_ref, o_ref):
    k = pl.program_id(1)

    @pl.when(k == 0)
    def _():
        o_ref[...] = jnp.dot(a_ref[...], w_ref[:784, :],
                             preferred_element_type=jnp.float32)

    koff = pl.multiple_of(k * 1024, 1024)
    o_ref[...] += jnp.dot(x_ref[:, pl.ds(koff, 1024)],
                          w_ref[pl.ds(784 + koff, 1024), :],
                          preferred_element_type=jnp.float32)

    @pl.when(k == pl.num_programs(1) - 1)
    def _():
        o_ref[...] = _bn_relu_epilogue(o_ref[...], b_ref, g_ref, be_ref)


def _layer1(a, xs, w1, b1, g1, be1):
    M, NB = 512, 256
    nn, nk = 4096 // NB, 7
    return pl.pallas_call(
        _l1_body,
        grid=(nn, nk),
        in_specs=[
            pl.BlockSpec((M, 784), lambda n, k: (0, 0)),
            pl.BlockSpec((M, 7168), lambda n, k: (0, 0)),
            pl.BlockSpec((7952, NB), lambda n, k: (0, n)),
            pl.BlockSpec((1, NB), lambda n, k: (0, n)),
            pl.BlockSpec((1, NB), lambda n, k: (0, n)),
            pl.BlockSpec((1, NB), lambda n, k: (0, n)),
        ],
        out_specs=pl.BlockSpec((M, NB), lambda n, k: (0, n)),
        out_shape=jax.ShapeDtypeStruct((M, 4096), jnp.float32),
        compiler_params=pltpu.CompilerParams(
            dimension_semantics=("parallel", "arbitrary")),
    )(a, xs, w1, b1, g1, be1)


def _l23_body(x_ref, w_ref, b_ref, g_ref, be_ref, o_ref, *, bn, kb):
    k = pl.program_id(1)

    @pl.when(k == 0)
    def _():
        o_ref[...] = jnp.zeros_like(o_ref)

    koff = pl.multiple_of(k * kb, kb)
    o_ref[...] += jnp.dot(x_ref[:, pl.ds(koff, kb)], w_ref[...],
                          preferred_element_type=jnp.float32)

    @pl.when(k == pl.num_programs(1) - 1)
    def _():
        if bn:
            o_ref[...] = _bn_relu_epilogue(o_ref[...], b_ref, g_ref, be_ref)
        else:
            o_ref[...] = o_ref[...] + b_ref[...]


def _layer23(h, w, b, g, be, bn):
    M, NB, KB = 512, 256, 1024
    K, NO = w.shape
    nn, nk = NO // NB, K // KB
    specs = [
        pl.BlockSpec((M, K), lambda n, k: (0, 0)),
        pl.BlockSpec((KB, NB), lambda n, k: (k, n)),
        pl.BlockSpec((1, NB), lambda n, k: (0, n)),
        pl.BlockSpec((1, NB), lambda n, k: (0, n)),
        pl.BlockSpec((1, NB), lambda n, k: (0, n)),
    ]
    return pl.pallas_call(
        functools.partial(_l23_body, bn=bn, kb=KB),
        grid=(nn, nk),
        in_specs=specs,
        out_specs=pl.BlockSpec((M, NB), lambda n, k: (0, n)),
        out_shape=jax.ShapeDtypeStruct((M, NO), jnp.float32),
        compiler_params=pltpu.CompilerParams(
            dimension_semantics=("parallel", "arbitrary")),
    )(h, w, b, g, be)


def kernel(x, edge_index, batch, W1, b1, g1, be1, W2, b2, g2, be2, W3, b3):
    x_aug = jnp.concatenate(
        [x, jnp.zeros((N_PAD_ROWS, D_FEAT), x.dtype)], axis=0)
    a_flat, xs = _densify(x_aug, edge_index, batch)
    a = a_flat.reshape(G, A_ROW)[:, :784]
    xrows = xs.reshape(G, MAX_NODES * D_FEAT)

    h1 = _layer1(a, xrows, W1, b1.reshape(1, -1), g1.reshape(1, -1),
                 be1.reshape(1, -1))
    h2 = _layer23(h1, W2, b2.reshape(1, -1), g2.reshape(1, -1),
                  be2.reshape(1, -1), bn=True)
    return _layer23(h2, W3, b3.reshape(1, -1), b3.reshape(1, -1),
                    b3.reshape(1, -1), bn=False)


# speedup vs baseline: 3.5496x; 1.5527x over previous
import dataclasses
import functools

import jax
import jax.numpy as jnp
from jax import lax
from jax.experimental import pallas as pl
from jax.experimental.pallas import tpu as pltpu
from jax.experimental.pallas import tpu_sc as plsc

MAX_NODES = 28
EPS = 1e-5
G = 512
N_NODES = 4096
E_EDGES = 32768
D_FEAT = 256
A_ROW = 1024
A_WORDS = G * A_ROW
X_ROWS = G * MAX_NODES
N_PAD_ROWS = 256

NC, NS = 2, 16
NW = NC * NS
XR_PER_TILE = X_ROWS // NW
X_CHUNK = 112
E_PER_TILE = E_EDGES // NS


def _densify_body(x_hbm, ei_hbm, batch_hbm, a_hbm, xs_hbm,
                  batch_v, counts_v, starts_v, rows_buf, xidx_v,
                  src_v, dst_v, eidx2d, ones_v, zbuf, a_sh, sem):
    cid = lax.axis_index("c")
    sid = lax.axis_index("s")
    wid = cid * NS + sid

    pltpu.sync_copy(batch_hbm, batch_v)

    @pl.loop(0, G // 16)
    def _zero_counts(j):
        counts_v[pl.ds(j * 16, 16)] = jnp.zeros((16,), jnp.int32)

    ones16 = jnp.ones((16,), jnp.int32)

    @pl.loop(0, N_NODES // 16)
    def _hist(j):
        b = batch_v[pl.ds(j * 16, 16)]
        plsc.addupdate_scatter(counts_v, [b], ones16)

    def _scan_body(j, carry):
        c = counts_v[pl.ds(j * 16, 16)]
        incl = plsc.cumsum(c)
        starts_v[pl.ds(j * 16, 16)] = incl - c + carry
        return carry + jnp.sum(c)

    lax.fori_loop(0, G // 16, _scan_body, jnp.int32(0))

    @pl.loop(0, 8192 // 16)
    def _zero_zbuf(j):
        zbuf[pl.ds(j * 16, 16)] = jnp.zeros((16,), jnp.float32)

    @pl.loop(0, 128 // 16)
    def _init_ones(j):
        ones_v[pl.ds(j * 16, 16)] = jnp.ones((16,), jnp.float32)

    @pl.when(cid == 0)
    def _zero_spmem():
        for q in range(4):
            pltpu.sync_copy(zbuf, a_sh.at[pl.ds(sid * 32768 + q * 8192, 8192)])

    plsc.subcore_barrier()

    @pl.when(cid == 0)
    def _edges():
        pltpu.sync_copy(ei_hbm.at[0, pl.ds(sid * E_PER_TILE, E_PER_TILE)],
                        src_v)
        pltpu.sync_copy(ei_hbm.at[1, pl.ds(sid * E_PER_TILE, E_PER_TILE)],
                        dst_v)

        @pl.loop(0, 16)
        def _row(r):
            @pl.loop(0, 8)
            def _chunk(cc):
                j = r * 8 + cc
                s = src_v[pl.ds(j * 16, 16)]
                d = dst_v[pl.ds(j * 16, 16)]
                g = plsc.load_gather(batch_v, [s])
                st = plsc.load_gather(starts_v, [g])
                ls = s - st
                ld = d - st
                valid = (ls < MAX_NODES) & (ld >= 0) & (ld < MAX_NODES)
                flat = g * A_ROW + ls * MAX_NODES + ld
                dump = (s & 511) * A_ROW + 784 + (d & 127)
                eidx2d[r, pl.ds(cc * 16, 16)] = jnp.where(valid, flat, dump)

            pltpu.sync_copy(ones_v, a_sh.at[eidx2d.at[r]], add=True)

    plsc.subcore_barrier()

    @pl.when(cid == 0)
    def _copy_a():
        pltpu.sync_copy(a_sh.at[pl.ds(sid * 32768, 32768)],
                        a_hbm.at[pl.ds(sid * 32768, 32768)])

    base = wid * XR_PER_TILE
    for c in range(XR_PER_TILE // X_CHUNK):
        cbase = base + c * X_CHUNK

        @pl.loop(0, X_CHUNK // 16)
        def _mkidx(i):
            r = cbase + i * 16 + lax.iota(jnp.int32, 16)
            gg = r // MAX_NODES
            ll = r - gg * MAX_NODES
            st = plsc.load_gather(starts_v, [gg])
            cnt = plsc.load_gather(counts_v, [gg])
            srcp = st + ll
            sentinel = N_NODES + (r & (N_PAD_ROWS - 1))
            xidx_v[pl.ds(i * 16, 16)] = jnp.where(ll < cnt, srcp, sentinel)

        pltpu.async_copy(x_hbm.at[xidx_v], rows_buf, sem).wait()
        pltpu.sync_copy(rows_buf, xs_hbm.at[pl.ds(cbase, X_CHUNK)])


def _sc_compiler_params():
    cp = pltpu.CompilerParams()
    if "needs_layout_passes" in pltpu.CompilerParams.__dataclass_fields__:
        cp = dataclasses.replace(cp, needs_layout_passes=False)
    return cp


def _densify(x_aug, edge_index, batch):
    mesh = plsc.VectorSubcoreMesh(core_axis_name="c", subcore_axis_name="s")
    kern = pl.kernel(
        _densify_body,
        out_type=(
            jax.ShapeDtypeStruct((A_WORDS,), jnp.float32),
            jax.ShapeDtypeStruct((X_ROWS, D_FEAT), jnp.float32),
        ),
        mesh=mesh,
        scratch_types=[
            pltpu.VMEM((N_NODES,), jnp.int32),
            pltpu.VMEM((G,), jnp.int32),
            pltpu.VMEM((G,), jnp.int32),
            pltpu.VMEM((X_CHUNK, D_FEAT), jnp.float32),
            pltpu.VMEM((X_CHUNK,), jnp.int32),
            pltpu.VMEM((E_PER_TILE,), jnp.int32),
            pltpu.VMEM((E_PER_TILE,), jnp.int32),
            pltpu.VMEM((16, 128), jnp.int32),
            pltpu.VMEM((128,), jnp.float32),
            pltpu.VMEM((8192,), jnp.float32),
            pltpu.VMEM_SHARED((A_WORDS,), jnp.float32),
            pltpu.SemaphoreType.DMA,
        ],
        compiler_params=_sc_compiler_params(),
    )
    return kern(x_aug, edge_index, batch)


def _bn_relu_epilogue(acc, b_ref, g_ref, be_ref):
    h = acc + b_ref[...]
    mu = jnp.mean(h, axis=0, keepdims=True)
    hc = h - mu
    var = jnp.mean(hc * hc, axis=0, keepdims=True)
    return jnp.maximum(hc * lax.rsqrt(var + EPS) * g_ref[...] + be_ref[...],
                       0.0)


def _l1_body(a_ref, x_ref, w_ref, b_ref, g_ref, be_ref, o_ref):
    acc = jnp.dot(a_ref[...], w_ref[:784, :],
                  preferred_element_type=jnp.float32)
    acc += jnp.dot(x_ref[...], w_ref[784:, :],
                   preferred_element_type=jnp.float32)
    o_ref[...] = _bn_relu_epilogue(acc, b_ref, g_ref, be_ref)


def _layer1(a, xs, w1, b1, g1, be1):
    M, NB = 512, 256
    nn = 4096 // NB
    return pl.pallas_call(
        _l1_body,
        grid=(nn,),
        in_specs=[
            pl.BlockSpec((M, 784), lambda n: (0, 0)),
            pl.BlockSpec((M, 7168), lambda n: (0, 0)),
            pl.BlockSpec((7952, NB), lambda n: (0, n)),
            pl.BlockSpec((1, NB), lambda n: (0, n)),
            pl.BlockSpec((1, NB), lambda n: (0, n)),
            pl.BlockSpec((1, NB), lambda n: (0, n)),
        ],
        out_specs=pl.BlockSpec((M, NB), lambda n: (0, n)),
        out_shape=jax.ShapeDtypeStruct((M, 4096), jnp.float32),
        compiler_params=pltpu.CompilerParams(
            dimension_semantics=("arbitrary",)),
    )(a, xs, w1, b1, g1, be1)


def _l23_body(x_ref, w_ref, b_ref, g_ref, be_ref, o_ref, *, bn):
    acc = jnp.dot(x_ref[...], w_ref[...], preferred_element_type=jnp.float32)
    if bn:
        o_ref[...] = _bn_relu_epilogue(acc, b_ref, g_ref, be_ref)
    else:
        o_ref[...] = acc + b_ref[...]


def _layer23(h, w, b, g, be, bn):
    M, NB = 512, 256
    K, NO = w.shape
    nn = NO // NB
    specs = [
        pl.BlockSpec((M, K), lambda n: (0, 0)),
        pl.BlockSpec((K, NB), lambda n: (0, n)),
        pl.BlockSpec((1, NB), lambda n: (0, n)),
        pl.BlockSpec((1, NB), lambda n: (0, n)),
        pl.BlockSpec((1, NB), lambda n: (0, n)),
    ]
    return pl.pallas_call(
        functools.partial(_l23_body, bn=bn),
        grid=(nn,),
        in_specs=specs,
        out_specs=pl.BlockSpec((M, NB), lambda n: (0, n)),
        out_shape=jax.ShapeDtypeStruct((M, NO), jnp.float32),
        compiler_params=pltpu.CompilerParams(
            dimension_semantics=("arbitrary",)),
    )(h, w, b, g, be)


def kernel(x, edge_index, batch, W1, b1, g1, be1, W2, b2, g2, be2, W3, b3):
    x_aug = jnp.concatenate(
        [x, jnp.zeros((N_PAD_ROWS, D_FEAT), x.dtype)], axis=0)
    a_flat, xs = _densify(x_aug, edge_index, batch)
    a = a_flat.reshape(G, A_ROW)[:, :784]
    xrows = xs.reshape(G, MAX_NODES * D_FEAT)

    h1 = _layer1(a, xrows, W1, b1.reshape(1, -1), g1.reshape(1, -1),
                 be1.reshape(1, -1))
    h2 = _layer23(h1, W2, b2.reshape(1, -1), g2.reshape(1, -1),
                  be2.reshape(1, -1), bn=True)
    return _layer23(h2, W3, b3.reshape(1, -1), b3.reshape(1, -1),
                    b3.reshape(1, -1), bn=False)


# speedup vs baseline: 3.5740x; 1.0069x over previous
import dataclasses
import functools

import jax
import jax.numpy as jnp
from jax import lax
from jax.experimental import pallas as pl
from jax.experimental.pallas import tpu as pltpu
from jax.experimental.pallas import tpu_sc as plsc

MAX_NODES = 28
EPS = 1e-5
G = 512
N_NODES = 4096
E_EDGES = 32768
D_FEAT = 256
A_ROW = 1024
A_WORDS = G * A_ROW
X_ROWS = G * MAX_NODES
N_PAD_ROWS = 256

NC, NS = 2, 16
NW = NC * NS
XR_PER_TILE = X_ROWS // NW
X_CHUNK = 112
E_PER_TILE = E_EDGES // NS


def _densify_body(x_hbm, ei_hbm, batch_hbm, a_hbm, xs_hbm,
                  batch_v, counts_v, starts_v, rows_buf, xidx_v,
                  src_v, dst_v, eidx2d, ones_v, zbuf, a_sh, sem):
    cid = lax.axis_index("c")
    sid = lax.axis_index("s")
    wid = cid * NS + sid

    pltpu.sync_copy(batch_hbm, batch_v)

    @pl.loop(0, G // 16)
    def _zero_counts(j):
        counts_v[pl.ds(j * 16, 16)] = jnp.zeros((16,), jnp.int32)

    ones16 = jnp.ones((16,), jnp.int32)

    @pl.loop(0, N_NODES // 16)
    def _hist(j):
        b = batch_v[pl.ds(j * 16, 16)]
        plsc.addupdate_scatter(counts_v, [b], ones16)

    def _scan_body(j, carry):
        c = counts_v[pl.ds(j * 16, 16)]
        incl = plsc.cumsum(c)
        starts_v[pl.ds(j * 16, 16)] = incl - c + carry
        return carry + jnp.sum(c)

    lax.fori_loop(0, G // 16, _scan_body, jnp.int32(0))

    @pl.loop(0, 8192 // 16)
    def _zero_zbuf(j):
        zbuf[pl.ds(j * 16, 16)] = jnp.zeros((16,), jnp.float32)

    @pl.loop(0, 128 // 16)
    def _init_ones(j):
        ones_v[pl.ds(j * 16, 16)] = jnp.ones((16,), jnp.float32)

    @pl.when(cid == 0)
    def _zero_spmem():
        for q in range(4):
            pltpu.sync_copy(zbuf, a_sh.at[pl.ds(sid * 32768 + q * 8192, 8192)])

    plsc.subcore_barrier()

    @pl.when(cid == 0)
    def _edges():
        pltpu.sync_copy(ei_hbm.at[0, pl.ds(sid * E_PER_TILE, E_PER_TILE)],
                        src_v)
        pltpu.sync_copy(ei_hbm.at[1, pl.ds(sid * E_PER_TILE, E_PER_TILE)],
                        dst_v)

        @pl.loop(0, 16)
        def _row(r):
            @pl.loop(0, 8)
            def _chunk(cc):
                j = r * 8 + cc
                s = src_v[pl.ds(j * 16, 16)]
                d = dst_v[pl.ds(j * 16, 16)]
                g = plsc.load_gather(batch_v, [s])
                st = plsc.load_gather(starts_v, [g])
                ls = s - st
                ld = d - st
                valid = (ls < MAX_NODES) & (ld >= 0) & (ld < MAX_NODES)
                flat = g * A_ROW + ls * MAX_NODES + ld
                dump = (s & 511) * A_ROW + 784 + (d & 127)
                eidx2d[r, pl.ds(cc * 16, 16)] = jnp.where(valid, flat, dump)

            pltpu.sync_copy(ones_v, a_sh.at[eidx2d.at[r]], add=True)

    plsc.subcore_barrier()

    @pl.when(cid == 0)
    def _copy_a():
        pltpu.sync_copy(a_sh.at[pl.ds(sid * 32768, 32768)],
                        a_hbm.at[pl.ds(sid * 32768, 32768)])

    base = wid * XR_PER_TILE
    for c in range(XR_PER_TILE // X_CHUNK):
        cbase = base + c * X_CHUNK

        @pl.loop(0, X_CHUNK // 16)
        def _mkidx(i):
            r = cbase + i * 16 + lax.iota(jnp.int32, 16)
            gg = r // MAX_NODES
            ll = r - gg * MAX_NODES
            st = plsc.load_gather(starts_v, [gg])
            cnt = plsc.load_gather(counts_v, [gg])
            srcp = st + ll
            sentinel = N_NODES + (r & (N_PAD_ROWS - 1))
            xidx_v[pl.ds(i * 16, 16)] = jnp.where(ll < cnt, srcp, sentinel)

        pltpu.async_copy(x_hbm.at[xidx_v], rows_buf, sem).wait()
        pltpu.sync_copy(rows_buf, xs_hbm.at[pl.ds(cbase, X_CHUNK)])


def _sc_compiler_params():
    cp = pltpu.CompilerParams()
    if "needs_layout_passes" in pltpu.CompilerParams.__dataclass_fields__:
        cp = dataclasses.replace(cp, needs_layout_passes=False)
    return cp


def _densify(x_aug, edge_index, batch):
    mesh = plsc.VectorSubcoreMesh(core_axis_name="c", subcore_axis_name="s")
    kern = pl.kernel(
        _densify_body,
        out_type=(
            jax.ShapeDtypeStruct((A_WORDS,), jnp.float32),
            jax.ShapeDtypeStruct((X_ROWS, D_FEAT), jnp.float32),
        ),
        mesh=mesh,
        scratch_types=[
            pltpu.VMEM((N_NODES,), jnp.int32),
            pltpu.VMEM((G,), jnp.int32),
            pltpu.VMEM((G,), jnp.int32),
            pltpu.VMEM((X_CHUNK, D_FEAT), jnp.float32),
            pltpu.VMEM((X_CHUNK,), jnp.int32),
            pltpu.VMEM((E_PER_TILE,), jnp.int32),
            pltpu.VMEM((E_PER_TILE,), jnp.int32),
            pltpu.VMEM((16, 128), jnp.int32),
            pltpu.VMEM((128,), jnp.float32),
            pltpu.VMEM((8192,), jnp.float32),
            pltpu.VMEM_SHARED((A_WORDS,), jnp.float32),
            pltpu.SemaphoreType.DMA,
        ],
        compiler_params=_sc_compiler_params(),
    )
    return kern(x_aug, edge_index, batch)


def _bn_relu_epilogue(acc, b_ref, g_ref, be_ref):
    h = acc + b_ref[...]
    mu = jnp.mean(h, axis=0, keepdims=True)
    hc = h - mu
    var = jnp.mean(hc * hc, axis=0, keepdims=True)
    return jnp.maximum(hc * lax.rsqrt(var + EPS) * g_ref[...] + be_ref[...],
                       0.0)


def _l1_body(a_ref, x_ref, w_ref, b_ref, g_ref, be_ref, o_ref):
    acc = jnp.dot(a_ref[...], w_ref[:784, :],
                  preferred_element_type=jnp.float32)
    acc += jnp.dot(x_ref[...], w_ref[784:, :],
                   preferred_element_type=jnp.float32)
    o_ref[...] = _bn_relu_epilogue(acc, b_ref, g_ref, be_ref
                                   ).astype(o_ref.dtype)


def _layer1(a, xs, w1, b1, g1, be1):
    M, NB = 512, 256
    nn = 4096 // NB
    return pl.pallas_call(
        _l1_body,
        grid=(nn,),
        in_specs=[
            pl.BlockSpec((M, 784), lambda n: (0, 0)),
            pl.BlockSpec((M, 7168), lambda n: (0, 0)),
            pl.BlockSpec((7952, NB), lambda n: (0, n)),
            pl.BlockSpec((1, NB), lambda n: (0, n)),
            pl.BlockSpec((1, NB), lambda n: (0, n)),
            pl.BlockSpec((1, NB), lambda n: (0, n)),
        ],
        out_specs=pl.BlockSpec((M, NB), lambda n: (0, n)),
        out_shape=jax.ShapeDtypeStruct((M, 4096), jnp.bfloat16),
        compiler_params=pltpu.CompilerParams(
            dimension_semantics=("arbitrary",)),
    )(a, xs, w1, b1, g1, be1)


def _l23_body(x_ref, w_ref, b_ref, g_ref, be_ref, o_ref, *, bn):
    acc = jnp.dot(x_ref[...], w_ref[...], preferred_element_type=jnp.float32)
    if bn:
        o_ref[...] = _bn_relu_epilogue(acc, b_ref, g_ref, be_ref
                                       ).astype(o_ref.dtype)
    else:
        o_ref[...] = acc + b_ref[...]


def _layer23(h, w, b, g, be, bn, out_dtype=jnp.float32):
    M, NB = 512, 256
    K, NO = w.shape
    nn = NO // NB
    specs = [
        pl.BlockSpec((M, K), lambda n: (0, 0)),
        pl.BlockSpec((K, NB), lambda n: (0, n)),
        pl.BlockSpec((1, NB), lambda n: (0, n)),
        pl.BlockSpec((1, NB), lambda n: (0, n)),
        pl.BlockSpec((1, NB), lambda n: (0, n)),
    ]
    return pl.pallas_call(
        functools.partial(_l23_body, bn=bn),
        grid=(nn,),
        in_specs=specs,
        out_specs=pl.BlockSpec((M, NB), lambda n: (0, n)),
        out_shape=jax.ShapeDtypeStruct((M, NO), out_dtype),
        compiler_params=pltpu.CompilerParams(
            dimension_semantics=("arbitrary",)),
    )(h, w, b, g, be)


def kernel(x, edge_index, batch, W1, b1, g1, be1, W2, b2, g2, be2, W3, b3):
    x_aug = jnp.concatenate(
        [x, jnp.zeros((N_PAD_ROWS, D_FEAT), x.dtype)], axis=0)
    a_flat, xs = _densify(x_aug, edge_index, batch)
    a = a_flat.reshape(G, A_ROW)[:, :784]
    xrows = xs.reshape(G, MAX_NODES * D_FEAT)

    h1 = _layer1(a.astype(jnp.bfloat16), xrows.astype(jnp.bfloat16), W1,
                 b1.reshape(1, -1), g1.reshape(1, -1), be1.reshape(1, -1))
    h2 = _layer23(h1, W2, b2.reshape(1, -1), g2.reshape(1, -1),
                  be2.reshape(1, -1), bn=True, out_dtype=jnp.bfloat16)
    return _layer23(h2, W3, b3.reshape(1, -1), b3.reshape(1, -1),
                    b3.reshape(1, -1), bn=False)


# speedup vs baseline: 3.6547x; 1.0226x over previous
import dataclasses
import functools

import jax
import jax.numpy as jnp
from jax import lax
from jax.experimental import pallas as pl
from jax.experimental.pallas import tpu as pltpu
from jax.experimental.pallas import tpu_sc as plsc

MAX_NODES = 28
EPS = 1e-5
G = 512
N_NODES = 4096
E_EDGES = 32768
D_FEAT = 256
A_ROW = 1024
A_WORDS = G * A_ROW
X_ROWS = G * MAX_NODES
N_PAD_ROWS = 256

NC, NS = 2, 16
NW = NC * NS
XR_PER_TILE = X_ROWS // NW
X_CHUNK = 112
E_PER_TILE = E_EDGES // NS


def _densify_body(x_hbm, ei_hbm, batch_hbm, a_hbm, xs_hbm,
                  batch_v, counts_v, starts_v, rows2, xidx2,
                  src_v, dst_v, eidx2d, ones_v, zbuf, a_sh,
                  gsem, osem, esem):
    cid = lax.axis_index("c")
    sid = lax.axis_index("s")
    wid = cid * NS + sid

    bcopy = pltpu.async_copy(batch_hbm, batch_v, esem)

    @pl.when(cid == 0)
    def _init_bufs():
        @pl.loop(0, 8192 // 16)
        def _zero_zbuf(j):
            zbuf[pl.ds(j * 16, 16)] = jnp.zeros((16,), jnp.float32)

        @pl.loop(0, 128 // 16)
        def _init_ones(j):
            ones_v[pl.ds(j * 16, 16)] = jnp.ones((16,), jnp.float32)

    @pl.loop(0, G // 16)
    def _zero_counts(j):
        counts_v[pl.ds(j * 16, 16)] = jnp.zeros((16,), jnp.int32)

    bcopy.wait()

    ones16 = jnp.ones((16,), jnp.int32)

    @pl.loop(0, N_NODES // 16)
    def _hist(j):
        b = batch_v[pl.ds(j * 16, 16)]
        plsc.addupdate_scatter(counts_v, [b], ones16)

    def _scan_body(j, carry):
        c = counts_v[pl.ds(j * 16, 16)]
        incl = plsc.cumsum(c)
        starts_v[pl.ds(j * 16, 16)] = incl - c + carry
        return carry + jnp.sum(c)

    lax.fori_loop(0, G // 16, _scan_body, jnp.int32(0))

    @pl.when(cid == 0)
    def _zero_spmem():
        zh = [pltpu.async_copy(
                  zbuf, a_sh.at[pl.ds(sid * 32768 + q * 8192, 8192)], esem)
              for q in range(4)]
        for h in zh:
            h.wait()

    plsc.subcore_barrier()

    @pl.when(cid == 0)
    def _edges():
        h1 = pltpu.async_copy(
            ei_hbm.at[0, pl.ds(sid * E_PER_TILE, E_PER_TILE)], src_v, esem)
        h2 = pltpu.async_copy(
            ei_hbm.at[1, pl.ds(sid * E_PER_TILE, E_PER_TILE)], dst_v, esem)
        h1.wait()
        h2.wait()

        hs = []
        for r in range(16):
            @pl.loop(0, 8)
            def _chunk(cc, r=r):
                j = r * 8 + cc
                s = src_v[pl.ds(j * 16, 16)]
                d = dst_v[pl.ds(j * 16, 16)]
                g = plsc.load_gather(batch_v, [s])
                st = plsc.load_gather(starts_v, [g])
                ls = s - st
                ld = d - st
                valid = (ls < MAX_NODES) & (ld >= 0) & (ld < MAX_NODES)
                flat = g * A_ROW + ls * MAX_NODES + ld
                dump = (s & 511) * A_ROW + 784 + (d & 127)
                eidx2d[r, pl.ds(cc * 16, 16)] = jnp.where(valid, flat, dump)

            hs.append(pltpu.async_copy(
                ones_v, a_sh.at[eidx2d.at[r]], esem, add=True))
        for h in hs:
            h.wait()

    plsc.subcore_barrier()

    @pl.when(cid == 0)
    def _copy_a():
        pltpu.async_copy(a_sh.at[pl.ds(sid * 32768, 32768)],
                         a_hbm.at[pl.ds(sid * 32768, 32768)], esem)

    base = wid * XR_PER_TILE
    nch = XR_PER_TILE // X_CHUNK

    def _mk_idx(c):
        cbase = base + c * X_CHUNK
        slot = c % 2

        @pl.loop(0, X_CHUNK // 16)
        def _mkidx(i):
            r = cbase + i * 16 + lax.iota(jnp.int32, 16)
            gg = r // MAX_NODES
            ll = r - gg * MAX_NODES
            st = plsc.load_gather(starts_v, [gg])
            cnt = plsc.load_gather(counts_v, [gg])
            srcp = st + ll
            sentinel = N_NODES + (r & (N_PAD_ROWS - 1))
            xidx2[slot, pl.ds(i * 16, 16)] = jnp.where(ll < cnt, srcp,
                                                       sentinel)

    def _start_gather(c):
        slot = c % 2
        return pltpu.async_copy(x_hbm.at[xidx2.at[slot]], rows2.at[slot],
                                gsem.at[slot])

    def _start_out(c):
        slot = c % 2
        return pltpu.async_copy(
            rows2.at[slot], xs_hbm.at[pl.ds(base + c * X_CHUNK, X_CHUNK)],
            osem.at[slot])

    _mk_idx(0)
    gh = {0: _start_gather(0)}
    oh = {}
    for c in range(nch):
        if c + 1 < nch:
            _mk_idx(c + 1)
            if c - 1 >= 0:
                oh[c - 1].wait()
            gh[c + 1] = _start_gather(c + 1)
        gh[c].wait()
        oh[c] = _start_out(c)
    oh[nch - 2].wait()
    oh[nch - 1].wait()

    @pl.when(cid == 0)
    def _drain_a():
        pltpu.make_async_copy(a_sh.at[pl.ds(sid * 32768, 32768)],
                              a_hbm.at[pl.ds(sid * 32768, 32768)],
                              esem).wait()


def _sc_compiler_params():
    cp = pltpu.CompilerParams()
    if "needs_layout_passes" in pltpu.CompilerParams.__dataclass_fields__:
        cp = dataclasses.replace(cp, needs_layout_passes=False)
    return cp


def _densify(x_aug, edge_index, batch):
    mesh = plsc.VectorSubcoreMesh(core_axis_name="c", subcore_axis_name="s")
    kern = pl.kernel(
        _densify_body,
        out_type=(
            jax.ShapeDtypeStruct((A_WORDS,), jnp.float32),
            jax.ShapeDtypeStruct((X_ROWS, D_FEAT), jnp.float32),
        ),
        mesh=mesh,
        scratch_types=[
            pltpu.VMEM((N_NODES,), jnp.int32),
            pltpu.VMEM((G,), jnp.int32),
            pltpu.VMEM((G,), jnp.int32),
            pltpu.VMEM((2, X_CHUNK, D_FEAT), jnp.float32),
            pltpu.VMEM((2, X_CHUNK), jnp.int32),
            pltpu.VMEM((E_PER_TILE,), jnp.int32),
            pltpu.VMEM((E_PER_TILE,), jnp.int32),
            pltpu.VMEM((16, 128), jnp.int32),
            pltpu.VMEM((128,), jnp.float32),
            pltpu.VMEM((8192,), jnp.float32),
            pltpu.VMEM_SHARED((A_WORDS,), jnp.float32),
            pltpu.SemaphoreType.DMA((2,)),
            pltpu.SemaphoreType.DMA((2,)),
            pltpu.SemaphoreType.DMA,
        ],
        compiler_params=_sc_compiler_params(),
    )
    return kern(x_aug, edge_index, batch)


def _bn_relu_epilogue(acc, b_ref, g_ref, be_ref):
    h = acc + b_ref[...]
    mu = jnp.mean(h, axis=0, keepdims=True)
    hc = h - mu
    var = jnp.mean(hc * hc, axis=0, keepdims=True)
    return jnp.maximum(hc * lax.rsqrt(var + EPS) * g_ref[...] + be_ref[...],
                       0.0)


def _l1_body(a_ref, x_ref, w_ref, b_ref, g_ref, be_ref, o_ref):
    acc = jnp.dot(a_ref[...], w_ref[:784, :],
                  preferred_element_type=jnp.float32)
    acc += jnp.dot(x_ref[...], w_ref[784:, :],
                   preferred_element_type=jnp.float32)
    o_ref[...] = _bn_relu_epilogue(acc, b_ref, g_ref, be_ref
                                   ).astype(o_ref.dtype)


def _layer1(a, xs, w1, b1, g1, be1):
    M, NB = 512, 256
    nn = 4096 // NB
    return pl.pallas_call(
        _l1_body,
        grid=(nn,),
        in_specs=[
            pl.BlockSpec((M, 784), lambda n: (0, 0)),
            pl.BlockSpec((M, 7168), lambda n: (0, 0)),
            pl.BlockSpec((7952, NB), lambda n: (0, n)),
            pl.BlockSpec((1, NB), lambda n: (0, n)),
            pl.BlockSpec((1, NB), lambda n: (0, n)),
            pl.BlockSpec((1, NB), lambda n: (0, n)),
        ],
        out_specs=pl.BlockSpec((M, NB), lambda n: (0, n)),
        out_shape=jax.ShapeDtypeStruct((M, 4096), jnp.bfloat16),
        compiler_params=pltpu.CompilerParams(
            dimension_semantics=("arbitrary",)),
    )(a, xs, w1, b1, g1, be1)


def _l23_body(x_ref, w_ref, b_ref, g_ref, be_ref, o_ref, *, bn):
    acc = jnp.dot(x_ref[...], w_ref[...], preferred_element_type=jnp.float32)
    if bn:
        o_ref[...] = _bn_relu_epilogue(acc, b_ref, g_ref, be_ref
                                       ).astype(o_ref.dtype)
    else:
        o_ref[...] = acc + b_ref[...]


def _layer23(h, w, b, g, be, bn, out_dtype=jnp.float32):
    M, NB = 512, 256
    K, NO = w.shape
    nn = NO // NB
    specs = [
        pl.BlockSpec((M, K), lambda n: (0, 0)),
        pl.BlockSpec((K, NB), lambda n: (0, n)),
        pl.BlockSpec((1, NB), lambda n: (0, n)),
        pl.BlockSpec((1, NB), lambda n: (0, n)),
        pl.BlockSpec((1, NB), lambda n: (0, n)),
    ]
    return pl.pallas_call(
        functools.partial(_l23_body, bn=bn),
        grid=(nn,),
        in_specs=specs,
        out_specs=pl.BlockSpec((M, NB), lambda n: (0, n)),
        out_shape=jax.ShapeDtypeStruct((M, NO), out_dtype),
        compiler_params=pltpu.CompilerParams(
            dimension_semantics=("arbitrary",)),
    )(h, w, b, g, be)


def kernel(x, edge_index, batch, W1, b1, g1, be1, W2, b2, g2, be2, W3, b3):
    x_aug = jnp.concatenate(
        [x, jnp.zeros((N_PAD_ROWS, D_FEAT), x.dtype)], axis=0)
    a_flat, xs = _densify(x_aug, edge_index, batch)
    a = a_flat.reshape(G, A_ROW)[:, :784]
    xrows = xs.reshape(G, MAX_NODES * D_FEAT)

    h1 = _layer1(a.astype(jnp.bfloat16), xrows.astype(jnp.bfloat16), W1,
                 b1.reshape(1, -1), g1.reshape(1, -1), be1.reshape(1, -1))
    h2 = _layer23(h1, W2, b2.reshape(1, -1), g2.reshape(1, -1),
                  be2.reshape(1, -1), bn=True, out_dtype=jnp.bfloat16)
    return _layer23(h2, W3, b3.reshape(1, -1), b3.reshape(1, -1),
                    b3.reshape(1, -1), bn=False)


# speedup vs baseline: 3.7101x; 1.0152x over previous
import dataclasses
import functools

import jax
import jax.numpy as jnp
from jax import lax
from jax.experimental import pallas as pl
from jax.experimental.pallas import tpu as pltpu
from jax.experimental.pallas import tpu_sc as plsc

MAX_NODES = 28
EPS = 1e-5
G = 512
N_NODES = 4096
E_EDGES = 32768
D_FEAT = 256
A_ROW = 1024
A_WORDS = G * A_ROW
X_ROWS = G * MAX_NODES
N_PAD_ROWS = 256

NC, NS = 2, 16
NW = NC * NS
XR_PER_TILE = X_ROWS // NW
X_CHUNK = 112
E_PER_TILE = E_EDGES // NS


def _densify_body(x_hbm, ei_hbm, batch_hbm, a_hbm, xs_hbm,
                  batch_v, counts_v, starts_v, rows2, xidx2,
                  src_v, dst_v, eidx2d, ones_v, zbuf, a_sh,
                  gsem, osem, esem):
    cid = lax.axis_index("c")
    sid = lax.axis_index("s")
    wid = cid * NS + sid

    bcopy = pltpu.async_copy(batch_hbm, batch_v, esem)

    @pl.when(cid == 0)
    def _init_bufs():
        @pl.loop(0, 8192 // 16)
        def _zero_zbuf(j):
            zbuf[pl.ds(j * 16, 16)] = jnp.zeros((16,), jnp.float32)

        @pl.loop(0, 128 // 16)
        def _init_ones(j):
            ones_v[pl.ds(j * 16, 16)] = jnp.ones((16,), jnp.float32)

    @pl.loop(0, G // 16)
    def _zero_counts(j):
        counts_v[pl.ds(j * 16, 16)] = jnp.zeros((16,), jnp.int32)

    bcopy.wait()

    ones16 = jnp.ones((16,), jnp.int32)

    @pl.loop(0, N_NODES // 16)
    def _hist(j):
        b = batch_v[pl.ds(j * 16, 16)]
        plsc.addupdate_scatter(counts_v, [b], ones16)

    def _scan_body(j, carry):
        c = counts_v[pl.ds(j * 16, 16)]
        incl = plsc.cumsum(c)
        starts_v[pl.ds(j * 16, 16)] = incl - c + carry
        return carry + jnp.sum(c)

    lax.fori_loop(0, G // 16, _scan_body, jnp.int32(0))

    @pl.when(cid == 0)
    def _zero_spmem():
        zh = [pltpu.async_copy(
                  zbuf, a_sh.at[pl.ds(sid * 32768 + q * 8192, 8192)], esem)
              for q in range(4)]
        for h in zh:
            h.wait()

    plsc.subcore_barrier()

    @pl.when(cid == 0)
    def _edges():
        h1 = pltpu.async_copy(
            ei_hbm.at[0, pl.ds(sid * E_PER_TILE, E_PER_TILE)], src_v, esem)
        h2 = pltpu.async_copy(
            ei_hbm.at[1, pl.ds(sid * E_PER_TILE, E_PER_TILE)], dst_v, esem)
        h1.wait()
        h2.wait()

        hs = []
        for r in range(16):
            @pl.loop(0, 8)
            def _chunk(cc, r=r):
                j = r * 8 + cc
                s = src_v[pl.ds(j * 16, 16)]
                d = dst_v[pl.ds(j * 16, 16)]
                g = plsc.load_gather(batch_v, [s])
                st = plsc.load_gather(starts_v, [g])
                ls = s - st
                ld = d - st
                valid = (ls < MAX_NODES) & (ld >= 0) & (ld < MAX_NODES)
                flat = g * A_ROW + ls * MAX_NODES + ld
                dump = (s & 511) * A_ROW + 784 + (d & 127)
                eidx2d[r, pl.ds(cc * 16, 16)] = jnp.where(valid, flat, dump)

            hs.append(pltpu.async_copy(
                ones_v, a_sh.at[eidx2d.at[r]], esem, add=True))
        for h in hs:
            h.wait()

    plsc.subcore_barrier()

    @pl.when(cid == 0)
    def _copy_a():
        pltpu.async_copy(a_sh.at[pl.ds(sid * 32768, 32768)],
                         a_hbm.at[pl.ds(sid * 32768, 32768)], esem)

    base = wid * XR_PER_TILE
    nch = XR_PER_TILE // X_CHUNK

    def _mk_idx(c):
        cbase = base + c * X_CHUNK
        slot = c % 2

        @pl.loop(0, X_CHUNK // 16)
        def _mkidx(i):
            r = cbase + i * 16 + lax.iota(jnp.int32, 16)
            gg = r // MAX_NODES
            ll = r - gg * MAX_NODES
            st = plsc.load_gather(starts_v, [gg])
            cnt = plsc.load_gather(counts_v, [gg])
            srcp = st + ll
            sentinel = N_NODES + (r & (N_PAD_ROWS - 1))
            xidx2[slot, pl.ds(i * 16, 16)] = jnp.where(ll < cnt, srcp,
                                                       sentinel)

    def _start_gather(c):
        slot = c % 2
        return pltpu.async_copy(x_hbm.at[xidx2.at[slot]], rows2.at[slot],
                                gsem.at[slot])

    def _start_out(c):
        slot = c % 2
        return pltpu.async_copy(
            rows2.at[slot], xs_hbm.at[pl.ds(base + c * X_CHUNK, X_CHUNK)],
            osem.at[slot])

    _mk_idx(0)
    gh = {0: _start_gather(0)}
    oh = {}
    for c in range(nch):
        if c + 1 < nch:
            _mk_idx(c + 1)
            if c - 1 >= 0:
                oh[c - 1].wait()
            gh[c + 1] = _start_gather(c + 1)
        gh[c].wait()
        oh[c] = _start_out(c)
    oh[nch - 2].wait()
    oh[nch - 1].wait()

    @pl.when(cid == 0)
    def _drain_a():
        pltpu.make_async_copy(a_sh.at[pl.ds(sid * 32768, 32768)],
                              a_hbm.at[pl.ds(sid * 32768, 32768)],
                              esem).wait()


def _sc_compiler_params():
    cp = pltpu.CompilerParams()
    if "needs_layout_passes" in pltpu.CompilerParams.__dataclass_fields__:
        cp = dataclasses.replace(cp, needs_layout_passes=False)
    return cp


def _densify(x_aug, edge_index, batch):
    mesh = plsc.VectorSubcoreMesh(core_axis_name="c", subcore_axis_name="s")
    kern = pl.kernel(
        _densify_body,
        out_type=(
            jax.ShapeDtypeStruct((A_WORDS,), jnp.float32),
            jax.ShapeDtypeStruct((X_ROWS, D_FEAT), jnp.float32),
        ),
        mesh=mesh,
        scratch_types=[
            pltpu.VMEM((N_NODES,), jnp.int32),
            pltpu.VMEM((G,), jnp.int32),
            pltpu.VMEM((G,), jnp.int32),
            pltpu.VMEM((2, X_CHUNK, D_FEAT), jnp.float32),
            pltpu.VMEM((2, X_CHUNK), jnp.int32),
            pltpu.VMEM((E_PER_TILE,), jnp.int32),
            pltpu.VMEM((E_PER_TILE,), jnp.int32),
            pltpu.VMEM((16, 128), jnp.int32),
            pltpu.VMEM((128,), jnp.float32),
            pltpu.VMEM((8192,), jnp.float32),
            pltpu.VMEM_SHARED((A_WORDS,), jnp.float32),
            pltpu.SemaphoreType.DMA((2,)),
            pltpu.SemaphoreType.DMA((2,)),
            pltpu.SemaphoreType.DMA,
        ],
        compiler_params=_sc_compiler_params(),
    )
    return kern(x_aug, edge_index, batch)


def _bn_relu_epilogue(acc, b_ref, g_ref, be_ref):
    h = acc + b_ref[...]
    mu = jnp.mean(h, axis=0, keepdims=True)
    hc = h - mu
    var = jnp.mean(hc * hc, axis=0, keepdims=True)
    return jnp.maximum(hc * lax.rsqrt(var + EPS) * g_ref[...] + be_ref[...],
                       0.0)


def _l1_body(a_ref, x_ref, w_ref, b_ref, g_ref, be_ref, o_ref):
    acc = jnp.dot(a_ref[...], w_ref[:784, :],
                  preferred_element_type=jnp.float32)
    acc += jnp.dot(x_ref[...], w_ref[784:, :],
                   preferred_element_type=jnp.float32)
    o_ref[...] = _bn_relu_epilogue(acc, b_ref, g_ref, be_ref
                                   ).astype(o_ref.dtype)


def _layer1(a, xs, w1, b1, g1, be1):
    M, NB = 512, 256
    nn = 4096 // NB
    return pl.pallas_call(
        _l1_body,
        grid=(nn,),
        in_specs=[
            pl.BlockSpec((M, 784), lambda n: (0, 0)),
            pl.BlockSpec((M, 7168), lambda n: (0, 0)),
            pl.BlockSpec((7952, NB), lambda n: (0, n)),
            pl.BlockSpec((1, NB), lambda n: (0, n)),
            pl.BlockSpec((1, NB), lambda n: (0, n)),
            pl.BlockSpec((1, NB), lambda n: (0, n)),
        ],
        out_specs=pl.BlockSpec((M, NB), lambda n: (0, n)),
        out_shape=jax.ShapeDtypeStruct((M, 4096), jnp.bfloat16),
        compiler_params=pltpu.CompilerParams(
            dimension_semantics=("arbitrary",)),
    )(a, xs, w1, b1, g1, be1)


def _l23_body(x_ref, w_ref, b_ref, g_ref, be_ref, o_ref, *, bn):
    acc = jnp.dot(x_ref[...], w_ref[...], preferred_element_type=jnp.float32)
    if bn:
        o_ref[...] = _bn_relu_epilogue(acc, b_ref, g_ref, be_ref
                                       ).astype(o_ref.dtype)
    else:
        o_ref[...] = acc + b_ref[...]


def _layer23(h, w, b, g, be, bn, out_dtype=jnp.float32):
    M, NB = 512, 256
    K, NO = w.shape
    nn = NO // NB
    specs = [
        pl.BlockSpec((M, K), lambda n: (0, 0)),
        pl.BlockSpec((K, NB), lambda n: (0, n)),
        pl.BlockSpec((1, NB), lambda n: (0, n)),
        pl.BlockSpec((1, NB), lambda n: (0, n)),
        pl.BlockSpec((1, NB), lambda n: (0, n)),
    ]
    return pl.pallas_call(
        functools.partial(_l23_body, bn=bn),
        grid=(nn,),
        in_specs=specs,
        out_specs=pl.BlockSpec((M, NB), lambda n: (0, n)),
        out_shape=jax.ShapeDtypeStruct((M, NO), out_dtype),
        compiler_params=pltpu.CompilerParams(
            dimension_semantics=("arbitrary",)),
    )(h, w, b, g, be)


def kernel(x, edge_index, batch, W1, b1, g1, be1, W2, b2, g2, be2, W3, b3):
    x_aug = jnp.concatenate(
        [x, jnp.zeros((N_PAD_ROWS, D_FEAT), x.dtype)], axis=0)
    a_flat, xs = _densify(x_aug, edge_index, batch)
    a = a_flat.reshape(G, A_ROW)[:, :784]
    xrows = xs.reshape(G, MAX_NODES * D_FEAT)

    h1 = _layer1(a, xrows, W1,
                 b1.reshape(1, -1), g1.reshape(1, -1), be1.reshape(1, -1))
    h2 = _layer23(h1, W2, b2.reshape(1, -1), g2.reshape(1, -1),
                  be2.reshape(1, -1), bn=True, out_dtype=jnp.bfloat16)
    return _layer23(h2, W3, b3.reshape(1, -1), b3.reshape(1, -1),
                    b3.reshape(1, -1), bn=False)
